# baseline (speedup 1.0000x reference)
import jax
import jax.numpy as jnp
from jax import lax
from jax.experimental import pallas as pl
from jax.experimental.pallas import tpu as pltpu
from jax.experimental.pallas import tpu_sc as plsc

SEQ = 198
DM = 512
BATCH = 32
JOINTS = 17
LANES = 16
BT, BI = 4, 8
DT, DI = 4, 128

SPLIT = 96
NWORK = 32
SC_SLABS = (SEQ - SPLIT) * JOINTS
NFULL = SC_SLABS // NWORK
NTAIL = SC_SLABS - NFULL * NWORK
DEPTH = 3
NGROUP = NFULL // DEPTH
P0 = SPLIT * JOINTS


def _newton_rsqrt(v):
    i = lax.bitcast_convert_type(v, jnp.int32)
    i = jnp.int32(0x5F3759DF) - (i >> 1)
    y = lax.bitcast_convert_type(i, jnp.float32)
    for _ in range(3):
        y = y * (jnp.float32(1.5) - jnp.float32(0.5) * v * y * y)
    return y


def _row_scale(erow_ref):
    acc = jnp.zeros((LANES,), jnp.float32)
    for c in range(DM // LANES):
        e = erow_ref[pl.ds(c * LANES, LANES)]
        acc = acc + e * e
    dnums = lax.GatherDimensionNumbers(
        offset_dims=(), collapsed_slice_dims=(0,), start_index_map=(0,))
    lane = lax.iota(jnp.int32, LANES)
    v = acc
    for sh in (1, 2, 4, 8):
        perm = (lane ^ jnp.int32(sh)).reshape(LANES, 1)
        v = v + lax.gather(v, perm, dnums, slice_sizes=(1,),
                           mode=lax.GatherScatterMode.PROMISE_IN_BOUNDS)
    return jnp.where(v > 1.0, _newton_rsqrt(v), jnp.float32(1.0))


def _add_slab(ib, ob, erow_ref):
    scale = _row_scale(erow_ref)
    ev = [[erow_ref[pl.ds(dt * DI + c * LANES, LANES)] * scale
           for c in range(DI // LANES)] for dt in range(DT)]

    @plsc.parallel_loop(0, BT * BI, unroll=4)
    def bibody(p):
        bt = p // BI
        bi = p % BI
        for dt in range(DT):
            for c in range(DI // LANES):
                ob[bt, dt, bi, pl.ds(c * LANES, LANES)] = (
                    ib[bt, dt, bi, pl.ds(c * LANES, LANES)] + ev[dt][c])


def _sc_body(x_hbm, tab_hbm, out_hbm,
             xin0, xin1, xin2, xout0, xout1, xout2, erow0, erow1, erow2,
             xsem, esem, osem):
    xin = [xin0, xin1, xin2]
    xout = [xout0, xout1, xout2]
    erow = [erow0, erow1, erow2]
    t = lax.axis_index("sub") * 2 + lax.axis_index("core")

    def sj(k):
        p = P0 + k * NWORK + t
        return p // JOINTS, p % JOINTS

    def start_loads(k, slot):
        s, j = sj(k)
        pltpu.make_async_copy(x_hbm.at[s, j], xin[slot],
                              xsem.at[slot]).start()
        pltpu.make_async_copy(tab_hbm.at[s], erow[slot],
                              esem.at[slot]).start()

    for j in range(DEPTH):
        start_loads(j, j)

    def gbody(g, carry):
        for j in range(DEPTH):
            k = g * DEPTH + j
            s, js = sj(k)

            @pl.when(g > 0)
            def _():
                ks, kj = sj(k - DEPTH)
                pltpu.make_async_copy(xout[j], out_hbm.at[ks, kj],
                                      osem.at[j]).wait()

            pltpu.make_async_copy(x_hbm.at[s, js], xin[j],
                                  xsem.at[j]).wait()
            pltpu.make_async_copy(tab_hbm.at[s], erow[j],
                                  esem.at[j]).wait()

            _add_slab(xin[j], xout[j], erow[j])

            pltpu.make_async_copy(xout[j], out_hbm.at[s, js],
                                  osem.at[j]).start()

            @pl.when(g < NGROUP - 1)
            def _():
                start_loads(k + DEPTH, j)
        return carry

    lax.fori_loop(0, NGROUP, gbody, 0)

    for j in range(DEPTH):
        s, js = sj(NFULL - DEPTH + j)
        pltpu.make_async_copy(xout[j], out_hbm.at[s, js],
                              osem.at[j]).wait()

    @pl.when(t < NTAIL)
    def _():
        p = P0 + NFULL * NWORK + t
        s, js = p // JOINTS, p % JOINTS
        pltpu.make_async_copy(x_hbm.at[s, js], xin0, xsem.at[0]).start()
        pltpu.make_async_copy(tab_hbm.at[s], erow0, esem.at[0]).start()
        pltpu.make_async_copy(x_hbm.at[s, js], xin0, xsem.at[0]).wait()
        pltpu.make_async_copy(tab_hbm.at[s], erow0, esem.at[0]).wait()
        _add_slab(xin0, xout0, erow0)
        pltpu.make_async_copy(xout0, out_hbm.at[s, js], osem.at[0]).start()
        pltpu.make_async_copy(xout0, out_hbm.at[s, js], osem.at[0]).wait()


def _sc_part(z, table):
    mesh = plsc.VectorSubcoreMesh(core_axis_name="core",
                                  subcore_axis_name="sub")
    return pl.kernel(
        _sc_body,
        out_type=jax.ShapeDtypeStruct((SEQ, JOINTS, BT, DT, BI, DI),
                                      jnp.float32),
        mesh=mesh,
        scratch_types=(
            [pltpu.VMEM((BT, DT, BI, DI), jnp.float32)] * 6
            + [pltpu.VMEM((DM,), jnp.float32)] * 3
            + [pltpu.SemaphoreType.DMA((DEPTH,))] * 3
        ),
    )(z, table)


TCROWS = 8


def _tc_body(x_ref, tab_ref, prev_ref, out_ref):
    i = pl.program_id(0)
    e = tab_ref[pl.ds(i * TCROWS, TCROWS), :]
    sumsq = jnp.sum(e * e, axis=1, keepdims=True)
    scale = jnp.where(sumsq > 1.0, lax.rsqrt(jnp.maximum(sumsq, 1e-30)),
                      jnp.float32(1.0))
    ev = (e * scale).reshape(TCROWS, 1, 1, DT, 1, DI)
    out_ref[...] = x_ref[...] + ev


def _tc_part(z, table, prev):
    return pl.pallas_call(
        _tc_body,
        grid=(SPLIT // TCROWS, 2),
        in_specs=[
            pl.BlockSpec((TCROWS, JOINTS, BT // 2, DT, BI, DI),
                         lambda i, j: (i, 0, j, 0, 0, 0)),
            pl.BlockSpec((SEQ, DM), lambda i, j: (0, 0)),
            pl.BlockSpec(memory_space=pl.ANY),
        ],
        out_specs=pl.BlockSpec((TCROWS, JOINTS, BT // 2, DT, BI, DI),
                               lambda i, j: (i, 0, j, 0, 0, 0)),
        out_shape=jax.ShapeDtypeStruct((SEQ, JOINTS, BT, DT, BI, DI),
                                       jnp.float32),
        input_output_aliases={2: 0},
    )(z, table, prev)


def kernel(x, table):
    z = x.reshape(BT, BI, SEQ, JOINTS, DT, DI).transpose(2, 3, 0, 4, 1, 5)
    out_sc = _sc_part(z, table)
    out6 = _tc_part(z, table, out_sc)
    return out6.transpose(2, 4, 0, 1, 3, 5).reshape(BATCH, SEQ, JOINTS, DM)


# speedup vs baseline: 1.0176x; 1.0176x over previous
import jax
import jax.numpy as jnp
from jax import lax
from jax.experimental import pallas as pl
from jax.experimental.pallas import tpu as pltpu
from jax.experimental.pallas import tpu_sc as plsc

SEQ = 198
DM = 512
BATCH = 32
JOINTS = 17
LANES = 16
BT, BI = 4, 8
DT, DI = 4, 128

SPLIT = 112
NWORK = 32
SC_SLABS = (SEQ - SPLIT) * JOINTS
NFULL = SC_SLABS // NWORK
NTAIL = SC_SLABS - NFULL * NWORK
DEPTH = 3
NGROUP = NFULL // DEPTH
P0 = SPLIT * JOINTS


def _newton_rsqrt(v):
    i = lax.bitcast_convert_type(v, jnp.int32)
    i = jnp.int32(0x5F3759DF) - (i >> 1)
    y = lax.bitcast_convert_type(i, jnp.float32)
    for _ in range(3):
        y = y * (jnp.float32(1.5) - jnp.float32(0.5) * v * y * y)
    return y


def _row_scale(erow_ref):
    acc = jnp.zeros((LANES,), jnp.float32)
    for c in range(DM // LANES):
        e = erow_ref[pl.ds(c * LANES, LANES)]
        acc = acc + e * e
    dnums = lax.GatherDimensionNumbers(
        offset_dims=(), collapsed_slice_dims=(0,), start_index_map=(0,))
    lane = lax.iota(jnp.int32, LANES)
    v = acc
    for sh in (1, 2, 4, 8):
        perm = (lane ^ jnp.int32(sh)).reshape(LANES, 1)
        v = v + lax.gather(v, perm, dnums, slice_sizes=(1,),
                           mode=lax.GatherScatterMode.PROMISE_IN_BOUNDS)
    return jnp.where(v > 1.0, _newton_rsqrt(v), jnp.float32(1.0))


def _add_slab(ib, ob, erow_ref):
    scale = _row_scale(erow_ref)
    ev = [[erow_ref[pl.ds(dt * DI + c * LANES, LANES)] * scale
           for c in range(DI // LANES)] for dt in range(DT)]

    @plsc.parallel_loop(0, BT * BI, unroll=4)
    def bibody(p):
        bt = p // BI
        bi = p % BI
        for dt in range(DT):
            for c in range(DI // LANES):
                ob[bt, dt, bi, pl.ds(c * LANES, LANES)] = (
                    ib[bt, dt, bi, pl.ds(c * LANES, LANES)] + ev[dt][c])


def _sc_body(x_hbm, tab_hbm, out_hbm,
             xin0, xin1, xin2, xout0, xout1, xout2, erow0, erow1, erow2,
             xsem, esem, osem):
    xin = [xin0, xin1, xin2]
    xout = [xout0, xout1, xout2]
    erow = [erow0, erow1, erow2]
    t = lax.axis_index("sub") * 2 + lax.axis_index("core")

    def sj(k):
        p = P0 + k * NWORK + t
        return p // JOINTS, p % JOINTS

    def start_loads(k, slot):
        s, j = sj(k)
        pltpu.make_async_copy(x_hbm.at[s, j], xin[slot],
                              xsem.at[slot]).start()
        pltpu.make_async_copy(tab_hbm.at[s], erow[slot],
                              esem.at[slot]).start()

    for j in range(DEPTH):
        start_loads(j, j)

    def gbody(g, carry):
        for j in range(DEPTH):
            k = g * DEPTH + j
            s, js = sj(k)

            @pl.when(g > 0)
            def _():
                ks, kj = sj(k - DEPTH)
                pltpu.make_async_copy(xout[j], out_hbm.at[ks, kj],
                                      osem.at[j]).wait()

            pltpu.make_async_copy(x_hbm.at[s, js], xin[j],
                                  xsem.at[j]).wait()
            pltpu.make_async_copy(tab_hbm.at[s], erow[j],
                                  esem.at[j]).wait()

            _add_slab(xin[j], xout[j], erow[j])

            pltpu.make_async_copy(xout[j], out_hbm.at[s, js],
                                  osem.at[j]).start()

            @pl.when(g < NGROUP - 1)
            def _():
                start_loads(k + DEPTH, j)
        return carry

    lax.fori_loop(0, NGROUP, gbody, 0)

    for j in range(DEPTH):
        s, js = sj(NFULL - DEPTH + j)
        pltpu.make_async_copy(xout[j], out_hbm.at[s, js],
                              osem.at[j]).wait()

    @pl.when(t < NTAIL)
    def _():
        p = P0 + NFULL * NWORK + t
        s, js = p // JOINTS, p % JOINTS
        pltpu.make_async_copy(x_hbm.at[s, js], xin0, xsem.at[0]).start()
        pltpu.make_async_copy(tab_hbm.at[s], erow0, esem.at[0]).start()
        pltpu.make_async_copy(x_hbm.at[s, js], xin0, xsem.at[0]).wait()
        pltpu.make_async_copy(tab_hbm.at[s], erow0, esem.at[0]).wait()
        _add_slab(xin0, xout0, erow0)
        pltpu.make_async_copy(xout0, out_hbm.at[s, js], osem.at[0]).start()
        pltpu.make_async_copy(xout0, out_hbm.at[s, js], osem.at[0]).wait()


def _sc_part(z, table):
    mesh = plsc.VectorSubcoreMesh(core_axis_name="core",
                                  subcore_axis_name="sub")
    return pl.kernel(
        _sc_body,
        out_type=jax.ShapeDtypeStruct((SEQ, JOINTS, BT, DT, BI, DI),
                                      jnp.float32),
        mesh=mesh,
        scratch_types=(
            [pltpu.VMEM((BT, DT, BI, DI), jnp.float32)] * 6
            + [pltpu.VMEM((DM,), jnp.float32)] * 3
            + [pltpu.SemaphoreType.DMA((DEPTH,))] * 3
        ),
    )(z, table)


TCROWS = 8


def _tc_body(x_ref, tab_ref, prev_ref, out_ref):
    i = pl.program_id(0)
    e = tab_ref[pl.ds(i * TCROWS, TCROWS), :]
    sumsq = jnp.sum(e * e, axis=1, keepdims=True)
    scale = jnp.where(sumsq > 1.0, lax.rsqrt(jnp.maximum(sumsq, 1e-30)),
                      jnp.float32(1.0))
    ev = (e * scale).reshape(TCROWS, 1, 1, DT, 1, DI)
    out_ref[...] = x_ref[...] + ev


def _tc_part(z, table, prev):
    return pl.pallas_call(
        _tc_body,
        grid=(SPLIT // TCROWS,),
        in_specs=[
            pl.BlockSpec((TCROWS, JOINTS, BT, DT, BI, DI),
                         lambda i: (i, 0, 0, 0, 0, 0)),
            pl.BlockSpec((SEQ, DM), lambda i: (0, 0)),
            pl.BlockSpec(memory_space=pl.ANY),
        ],
        out_specs=pl.BlockSpec((TCROWS, JOINTS, BT, DT, BI, DI),
                               lambda i: (i, 0, 0, 0, 0, 0)),
        out_shape=jax.ShapeDtypeStruct((SEQ, JOINTS, BT, DT, BI, DI),
                                       jnp.float32),
        input_output_aliases={2: 0},
    )(z, table, prev)


def kernel(x, table):
    z = x.reshape(BT, BI, SEQ, JOINTS, DT, DI).transpose(2, 3, 0, 4, 1, 5)
    out_sc = _sc_part(z, table)
    out6 = _tc_part(z, table, out_sc)
    return out6.transpose(2, 4, 0, 1, 3, 5).reshape(BATCH, SEQ, JOINTS, DM)
